# Initial kernel scaffold; baseline (speedup 1.0000x reference)
import jax
import jax.numpy as jnp
from jax.experimental import pallas as pl


def kernel(x, edge_index, edge_type, rel_emb, W_self, W_forward, W_rel, bias):
    raise NotImplementedError("write your pallas kernel here")


# speedup vs baseline: 4.8192x; 4.8192x over previous
import jax
import jax.numpy as jnp
from jax import lax
from jax.experimental import pallas as pl
from jax.experimental.pallas import tpu as pltpu
from jax.experimental.pallas import tpu_sc as plsc

N_NODES = 10000
N_EDGES = 320000
D = 128
N_REL = 256

NC = 2
NS = 16
NW = NC * NS
EPW = N_EDGES // NW
CHUNK = 80
NCHUNK = EPW // CHUNK
RELPT = N_REL // NS
ROWS_BIG = 640
ROWS_LAST = N_NODES - (NS - 1) * ROWS_BIG


def _sc_body(x_hbm, src_hbm, dst_hbm, et_hbm, rel_hbm, out_hbm,
             srcc, etc_, dstc, xbuf, rbuf, acc, negrel, sem1, sem2):
    c = lax.axis_index("c")
    s = lax.axis_index("s")
    w = c * NS + s

    def _zero_row(i, _):
        for k in range(D // 16):
            xbuf[i, pl.ds(k * 16, 16)] = jnp.zeros((16,), jnp.float32)
        return 0
    lax.fori_loop(0, CHUNK, _zero_row, 0)

    @pl.when(s < NS - 1)
    def _():
        for k in range(ROWS_BIG // CHUNK):
            off = pl.multiple_of(s * ROWS_BIG + k * CHUNK, 8)
            pltpu.sync_copy(xbuf, acc.at[pl.ds(off, CHUNK)])

    @pl.when(s == NS - 1)
    def _():
        for k in range(ROWS_LAST // CHUNK):
            pltpu.sync_copy(
                xbuf, acc.at[pl.ds((NS - 1) * ROWS_BIG + k * CHUNK, CHUNK)])

    roff = pl.multiple_of(s * RELPT, 8)
    pltpu.sync_copy(rel_hbm.at[pl.ds(roff, RELPT)], xbuf.at[pl.ds(0, RELPT)])
    for i in range(RELPT):
        for k in range(D // 16):
            xbuf[i, pl.ds(k * 16, 16)] = -xbuf[i, pl.ds(k * 16, 16)]
    pltpu.sync_copy(xbuf.at[pl.ds(0, RELPT)], negrel.at[pl.ds(roff, RELPT)])

    plsc.subcore_barrier()

    def _chunk(j, _):
        eo = pl.multiple_of(w * EPW + j * CHUNK, 8)
        pltpu.sync_copy(src_hbm.at[pl.ds(eo, CHUNK)], srcc)
        pltpu.sync_copy(dst_hbm.at[pl.ds(eo, CHUNK)], dstc)
        pltpu.sync_copy(et_hbm.at[pl.ds(eo, CHUNK)], etc_)
        pltpu.async_copy(x_hbm.at[srcc], xbuf, sem1).wait()
        pltpu.sync_copy(xbuf, acc.at[dstc], add=True)
        pltpu.async_copy(negrel.at[etc_], rbuf, sem2).wait()
        pltpu.sync_copy(rbuf, acc.at[dstc], add=True)
        return 0
    lax.fori_loop(0, NCHUNK, _chunk, 0)

    plsc.subcore_barrier()

    @pl.when(s < NS - 1)
    def _():
        off = pl.multiple_of(s * ROWS_BIG, 8)
        pltpu.sync_copy(acc.at[pl.ds(off, ROWS_BIG)],
                        out_hbm.at[c, pl.ds(off, ROWS_BIG)])

    @pl.when(s == NS - 1)
    def _():
        off = (NS - 1) * ROWS_BIG
        pltpu.sync_copy(acc.at[pl.ds(off, ROWS_LAST)],
                        out_hbm.at[c, pl.ds(off, ROWS_LAST)])


_sc_agg = pl.kernel(
    _sc_body,
    mesh=plsc.VectorSubcoreMesh(core_axis_name="c", subcore_axis_name="s"),
    out_type=jax.ShapeDtypeStruct((NC, N_NODES, D), jnp.float32),
    scratch_types=[
        pltpu.VMEM((CHUNK,), jnp.int32),
        pltpu.VMEM((CHUNK,), jnp.int32),
        pltpu.VMEM((CHUNK,), jnp.int32),
        pltpu.VMEM((CHUNK, D), jnp.float32),
        pltpu.VMEM((CHUNK, D), jnp.float32),
        pltpu.VMEM_SHARED((N_NODES, D), jnp.float32),
        pltpu.VMEM_SHARED((N_REL, D), jnp.float32),
        pltpu.SemaphoreType.DMA,
        pltpu.SemaphoreType.DMA,
    ],
)


BN = 2000


def _tc_body(x_ref, part_ref, rel_ref, ws_ref, wf_ref, wr_ref, b_ref,
             out_ref, relout_ref):
    i = pl.program_id(0)
    dn = (((1,), (1,)), ((), ()))
    agg = part_ref[0] + part_ref[1]
    out_ref[...] = (
        lax.dot_general(x_ref[...], ws_ref[...], dn,
                        preferred_element_type=jnp.float32)
        + lax.dot_general(agg, wf_ref[...], dn,
                          preferred_element_type=jnp.float32)
        + b_ref[...]
    )

    @pl.when(i == 0)
    def _():
        relout_ref[...] = lax.dot_general(rel_ref[...], wr_ref[...], dn,
                                          preferred_element_type=jnp.float32)


def _tc_finish(x, part, rel_emb, W_self, W_forward, W_rel, bias2d):
    return pl.pallas_call(
        _tc_body,
        grid=(N_NODES // BN,),
        in_specs=[
            pl.BlockSpec((BN, D), lambda i: (i, 0)),
            pl.BlockSpec((NC, BN, D), lambda i: (0, i, 0)),
            pl.BlockSpec((N_REL, D), lambda i: (0, 0)),
            pl.BlockSpec((D, D), lambda i: (0, 0)),
            pl.BlockSpec((D, D), lambda i: (0, 0)),
            pl.BlockSpec((D, D), lambda i: (0, 0)),
            pl.BlockSpec((1, D), lambda i: (0, 0)),
        ],
        out_specs=[
            pl.BlockSpec((BN, D), lambda i: (i, 0)),
            pl.BlockSpec((N_REL, D), lambda i: (0, 0)),
        ],
        out_shape=[
            jax.ShapeDtypeStruct((N_NODES, D), jnp.float32),
            jax.ShapeDtypeStruct((N_REL, D), jnp.float32),
        ],
    )(x, part, rel_emb, W_self, W_forward, W_rel, bias2d)


def kernel(x, edge_index, edge_type, rel_emb, W_self, W_forward, W_rel, bias):
    part = _sc_agg(x, edge_index[0], edge_index[1], edge_type, rel_emb)
    out, rel_out = _tc_finish(x, part, rel_emb, W_self, W_forward, W_rel,
                              bias.reshape(1, D))
    return out, rel_out


# speedup vs baseline: 7.6246x; 1.5821x over previous
import jax
import jax.numpy as jnp
from jax import lax
from jax.experimental import pallas as pl
from jax.experimental.pallas import tpu as pltpu
from jax.experimental.pallas import tpu_sc as plsc

N_NODES = 10000
N_EDGES = 320000
D = 128
N_REL = 256

NC = 2
NS = 16
NW = NC * NS
EPW = N_EDGES // NW
CHUNK = 80
NCHUNK = EPW // CHUNK
NBATCH = 5
BCH = NCHUNK // NBATCH
RELPT = N_REL // NS
ROWS_BIG = 640
ROWS_LAST = N_NODES - (NS - 1) * ROWS_BIG


def _sc_body(x_hbm, src_hbm, dst_hbm, et_hbm, rel_hbm, out_hbm,
             srcb, etb, dstb, xbuf, rbuf, acc, negrel, semx, semr):
    c = lax.axis_index("c")
    s = lax.axis_index("s")
    w = c * NS + s

    def _zero_row(i, _):
        for k in range(D // 16):
            xbuf[0, i, pl.ds(k * 16, 16)] = jnp.zeros((16,), jnp.float32)
        return 0
    lax.fori_loop(0, CHUNK, _zero_row, 0)
    zbuf = xbuf.at[0]

    @pl.when(s < NS - 1)
    def _():
        for k in range(ROWS_BIG // CHUNK):
            off = pl.multiple_of(s * ROWS_BIG + k * CHUNK, 8)
            pltpu.sync_copy(zbuf, acc.at[pl.ds(off, CHUNK)])

    @pl.when(s == NS - 1)
    def _():
        for k in range(ROWS_LAST // CHUNK):
            pltpu.sync_copy(
                zbuf, acc.at[pl.ds((NS - 1) * ROWS_BIG + k * CHUNK, CHUNK)])

    roff = pl.multiple_of(s * RELPT, 8)
    pltpu.sync_copy(rel_hbm.at[pl.ds(roff, RELPT)], zbuf.at[pl.ds(0, RELPT)])
    for i in range(RELPT):
        for k in range(D // 16):
            xbuf[0, i, pl.ds(k * 16, 16)] = -xbuf[0, i, pl.ds(k * 16, 16)]
    pltpu.sync_copy(zbuf.at[pl.ds(0, RELPT)], negrel.at[pl.ds(roff, RELPT)])

    plsc.subcore_barrier()

    def _batch(b, _):
        pltpu.sync_copy(src_hbm.at[w, b], srcb)
        pltpu.sync_copy(dst_hbm.at[w, b], dstb)
        pltpu.sync_copy(et_hbm.at[w, b], etb)
        pltpu.async_copy(x_hbm.at[srcb.at[0]], xbuf.at[0], semx.at[0])

        def _chunk(jj, _):
            p = lax.rem(jj, 2)
            q = 1 - p

            @pl.when(jj < BCH - 1)
            def _():
                pltpu.async_copy(x_hbm.at[srcb.at[jj + 1]], xbuf.at[q],
                                 semx.at[q])

            pltpu.make_async_copy(x_hbm.at[srcb.at[jj]], xbuf.at[p],
                                  semx.at[p]).wait()
            rg = pltpu.async_copy(negrel.at[etb.at[jj]], rbuf, semr)
            pltpu.sync_copy(xbuf.at[p], acc.at[dstb.at[jj]], add=True)
            rg.wait()
            pltpu.sync_copy(rbuf, acc.at[dstb.at[jj]], add=True)
            return 0
        lax.fori_loop(0, BCH, _chunk, 0)
        return 0
    lax.fori_loop(0, NBATCH, _batch, 0)

    plsc.subcore_barrier()

    @pl.when(s < NS - 1)
    def _():
        off = pl.multiple_of(s * ROWS_BIG, 8)
        pltpu.sync_copy(acc.at[pl.ds(off, ROWS_BIG)],
                        out_hbm.at[c, pl.ds(off, ROWS_BIG)])

    @pl.when(s == NS - 1)
    def _():
        off = (NS - 1) * ROWS_BIG
        pltpu.sync_copy(acc.at[pl.ds(off, ROWS_LAST)],
                        out_hbm.at[c, pl.ds(off, ROWS_LAST)])


_sc_agg = pl.kernel(
    _sc_body,
    mesh=plsc.VectorSubcoreMesh(core_axis_name="c", subcore_axis_name="s"),
    out_type=jax.ShapeDtypeStruct((NC, N_NODES, D), jnp.float32),
    scratch_types=[
        pltpu.VMEM((BCH, CHUNK), jnp.int32),
        pltpu.VMEM((BCH, CHUNK), jnp.int32),
        pltpu.VMEM((BCH, CHUNK), jnp.int32),
        pltpu.VMEM((2, CHUNK, D), jnp.float32),
        pltpu.VMEM((CHUNK, D), jnp.float32),
        pltpu.VMEM_SHARED((N_NODES, D), jnp.float32),
        pltpu.VMEM_SHARED((N_REL, D), jnp.float32),
        pltpu.SemaphoreType.DMA((2,)),
        pltpu.SemaphoreType.DMA,
    ],
)


BN = 2000


def _tc_body(x_ref, part_ref, rel_ref, ws_ref, wf_ref, wr_ref, b_ref,
             out_ref, relout_ref):
    i = pl.program_id(0)
    dn = (((1,), (1,)), ((), ()))
    agg = part_ref[0] + part_ref[1]
    out_ref[...] = (
        lax.dot_general(x_ref[...], ws_ref[...], dn,
                        preferred_element_type=jnp.float32)
        + lax.dot_general(agg, wf_ref[...], dn,
                          preferred_element_type=jnp.float32)
        + b_ref[...]
    )

    @pl.when(i == 0)
    def _():
        relout_ref[...] = lax.dot_general(rel_ref[...], wr_ref[...], dn,
                                          preferred_element_type=jnp.float32)


def _tc_finish(x, part, rel_emb, W_self, W_forward, W_rel, bias2d):
    return pl.pallas_call(
        _tc_body,
        grid=(N_NODES // BN,),
        in_specs=[
            pl.BlockSpec((BN, D), lambda i: (i, 0)),
            pl.BlockSpec((NC, BN, D), lambda i: (0, i, 0)),
            pl.BlockSpec((N_REL, D), lambda i: (0, 0)),
            pl.BlockSpec((D, D), lambda i: (0, 0)),
            pl.BlockSpec((D, D), lambda i: (0, 0)),
            pl.BlockSpec((D, D), lambda i: (0, 0)),
            pl.BlockSpec((1, D), lambda i: (0, 0)),
        ],
        out_specs=[
            pl.BlockSpec((BN, D), lambda i: (i, 0)),
            pl.BlockSpec((N_REL, D), lambda i: (0, 0)),
        ],
        out_shape=[
            jax.ShapeDtypeStruct((N_NODES, D), jnp.float32),
            jax.ShapeDtypeStruct((N_REL, D), jnp.float32),
        ],
    )(x, part, rel_emb, W_self, W_forward, W_rel, bias2d)


def kernel(x, edge_index, edge_type, rel_emb, W_self, W_forward, W_rel, bias):
    e4 = (NW, NBATCH, BCH, CHUNK)
    part = _sc_agg(x, edge_index[0].reshape(e4), edge_index[1].reshape(e4),
                   edge_type.reshape(e4), rel_emb)
    out, rel_out = _tc_finish(x, part, rel_emb, W_self, W_forward, W_rel,
                              bias.reshape(1, D))
    return out, rel_out


# speedup vs baseline: 7.6470x; 1.0029x over previous
import jax
import jax.numpy as jnp
from jax import lax
from jax.experimental import pallas as pl
from jax.experimental.pallas import tpu as pltpu
from jax.experimental.pallas import tpu_sc as plsc

N_NODES = 10000
N_EDGES = 320000
D = 128
N_REL = 256

NC = 2
NS = 16
NW = NC * NS
EPW = N_EDGES // NW
CHUNK = 80
NCHUNK = EPW // CHUNK
NBATCH = 5
BCH = NCHUNK // NBATCH
RELPT = N_REL // NS
ROWS_BIG = 640
ROWS_LAST = N_NODES - (NS - 1) * ROWS_BIG


def _sc_body(x_hbm, src_hbm, dst_hbm, et_hbm, rel_hbm, out_hbm,
             srcb, etb, dstb, xbuf, rbuf, acc, negrel, semx, semsx, semr):
    c = lax.axis_index("c")
    s = lax.axis_index("s")
    w = c * NS + s

    def _zero_row(i, _):
        for k in range(D // 16):
            xbuf[0, i, pl.ds(k * 16, 16)] = jnp.zeros((16,), jnp.float32)
        return 0
    lax.fori_loop(0, CHUNK, _zero_row, 0)
    zbuf = xbuf.at[0]

    @pl.when(s < NS - 1)
    def _():
        for k in range(ROWS_BIG // CHUNK):
            off = pl.multiple_of(s * ROWS_BIG + k * CHUNK, 8)
            pltpu.sync_copy(zbuf, acc.at[pl.ds(off, CHUNK)])

    @pl.when(s == NS - 1)
    def _():
        for k in range(ROWS_LAST // CHUNK):
            pltpu.sync_copy(
                zbuf, acc.at[pl.ds((NS - 1) * ROWS_BIG + k * CHUNK, CHUNK)])

    roff = pl.multiple_of(s * RELPT, 8)
    pltpu.sync_copy(rel_hbm.at[pl.ds(roff, RELPT)], zbuf.at[pl.ds(0, RELPT)])
    for i in range(RELPT):
        for k in range(D // 16):
            xbuf[0, i, pl.ds(k * 16, 16)] = -xbuf[0, i, pl.ds(k * 16, 16)]
    pltpu.sync_copy(zbuf.at[pl.ds(0, RELPT)], negrel.at[pl.ds(roff, RELPT)])

    plsc.subcore_barrier()

    def _batch(b, _):
        pltpu.sync_copy(src_hbm.at[w, b], srcb)
        pltpu.sync_copy(dst_hbm.at[w, b], dstb)
        pltpu.sync_copy(et_hbm.at[w, b], etb)
        pltpu.async_copy(x_hbm.at[srcb.at[0]], xbuf.at[0], semx.at[0])

        def _chunk(jj, _):
            p = lax.rem(jj, 2)
            q = 1 - p
            rg = pltpu.async_copy(negrel.at[etb.at[jj]], rbuf, semr)

            @pl.when(jj > 0)
            def _():
                pltpu.make_async_copy(xbuf.at[q], acc.at[dstb.at[jj - 1]],
                                      semsx.at[q]).wait()

            @pl.when(jj < BCH - 1)
            def _():
                pltpu.async_copy(x_hbm.at[srcb.at[jj + 1]], xbuf.at[q],
                                 semx.at[q])

            pltpu.make_async_copy(x_hbm.at[srcb.at[jj]], xbuf.at[p],
                                  semx.at[p]).wait()
            pltpu.async_copy(xbuf.at[p], acc.at[dstb.at[jj]], semsx.at[p],
                             add=True)
            rg.wait()
            pltpu.sync_copy(rbuf, acc.at[dstb.at[jj]], add=True)
            return 0
        lax.fori_loop(0, BCH, _chunk, 0)
        pltpu.make_async_copy(xbuf.at[(BCH - 1) % 2],
                              acc.at[dstb.at[BCH - 1]],
                              semsx.at[(BCH - 1) % 2]).wait()
        return 0
    lax.fori_loop(0, NBATCH, _batch, 0)

    plsc.subcore_barrier()

    @pl.when(s < NS - 1)
    def _():
        off = pl.multiple_of(s * ROWS_BIG, 8)
        pltpu.sync_copy(acc.at[pl.ds(off, ROWS_BIG)],
                        out_hbm.at[c, pl.ds(off, ROWS_BIG)])

    @pl.when(s == NS - 1)
    def _():
        off = (NS - 1) * ROWS_BIG
        pltpu.sync_copy(acc.at[pl.ds(off, ROWS_LAST)],
                        out_hbm.at[c, pl.ds(off, ROWS_LAST)])


_sc_agg = pl.kernel(
    _sc_body,
    mesh=plsc.VectorSubcoreMesh(core_axis_name="c", subcore_axis_name="s"),
    out_type=jax.ShapeDtypeStruct((NC, N_NODES, D), jnp.float32),
    scratch_types=[
        pltpu.VMEM((BCH, CHUNK), jnp.int32),
        pltpu.VMEM((BCH, CHUNK), jnp.int32),
        pltpu.VMEM((BCH, CHUNK), jnp.int32),
        pltpu.VMEM((2, CHUNK, D), jnp.float32),
        pltpu.VMEM((CHUNK, D), jnp.float32),
        pltpu.VMEM_SHARED((N_NODES, D), jnp.float32),
        pltpu.VMEM_SHARED((N_REL, D), jnp.float32),
        pltpu.SemaphoreType.DMA((2,)),
        pltpu.SemaphoreType.DMA((2,)),
        pltpu.SemaphoreType.DMA,
    ],
)


BN = 2000


def _tc_body(x_ref, part_ref, rel_ref, ws_ref, wf_ref, wr_ref, b_ref,
             out_ref, relout_ref):
    i = pl.program_id(0)
    dn = (((1,), (1,)), ((), ()))
    agg = part_ref[0] + part_ref[1]
    out_ref[...] = (
        lax.dot_general(x_ref[...], ws_ref[...], dn,
                        preferred_element_type=jnp.float32)
        + lax.dot_general(agg, wf_ref[...], dn,
                          preferred_element_type=jnp.float32)
        + b_ref[...]
    )

    @pl.when(i == 0)
    def _():
        relout_ref[...] = lax.dot_general(rel_ref[...], wr_ref[...], dn,
                                          preferred_element_type=jnp.float32)


def _tc_finish(x, part, rel_emb, W_self, W_forward, W_rel, bias2d):
    return pl.pallas_call(
        _tc_body,
        grid=(N_NODES // BN,),
        in_specs=[
            pl.BlockSpec((BN, D), lambda i: (i, 0)),
            pl.BlockSpec((NC, BN, D), lambda i: (0, i, 0)),
            pl.BlockSpec((N_REL, D), lambda i: (0, 0)),
            pl.BlockSpec((D, D), lambda i: (0, 0)),
            pl.BlockSpec((D, D), lambda i: (0, 0)),
            pl.BlockSpec((D, D), lambda i: (0, 0)),
            pl.BlockSpec((1, D), lambda i: (0, 0)),
        ],
        out_specs=[
            pl.BlockSpec((BN, D), lambda i: (i, 0)),
            pl.BlockSpec((N_REL, D), lambda i: (0, 0)),
        ],
        out_shape=[
            jax.ShapeDtypeStruct((N_NODES, D), jnp.float32),
            jax.ShapeDtypeStruct((N_REL, D), jnp.float32),
        ],
    )(x, part, rel_emb, W_self, W_forward, W_rel, bias2d)


def kernel(x, edge_index, edge_type, rel_emb, W_self, W_forward, W_rel, bias):
    e4 = (NW, NBATCH, BCH, CHUNK)
    part = _sc_agg(x, edge_index[0].reshape(e4), edge_index[1].reshape(e4),
                   edge_type.reshape(e4), rel_emb)
    out, rel_out = _tc_finish(x, part, rel_emb, W_self, W_forward, W_rel,
                              bias.reshape(1, D))
    return out, rel_out


# speedup vs baseline: 7.8884x; 1.0316x over previous
import jax
import jax.numpy as jnp
from jax import lax
from jax.experimental import pallas as pl
from jax.experimental.pallas import tpu as pltpu
from jax.experimental.pallas import tpu_sc as plsc

N_NODES = 10000
N_EDGES = 320000
D = 128
N_REL = 256

NC = 2
NS = 16
NW = NC * NS
EPW = N_EDGES // NW
CHUNK = 80
NCHUNK = EPW // CHUNK
NBATCH = 5
BCH = NCHUNK // NBATCH
HC = CHUNK // 2
RELPT = N_REL // NS
ROWS_BIG = 640
ROWS_LAST = N_NODES - (NS - 1) * ROWS_BIG


def _sc_body(x_hbm, src_hbm, dst_hbm, et_hbm, rel_hbm, out_hbm,
             srcb, etb, dstb, xbuf, rbuf, acc, negrel, semx, semsx, semr,
             semrs):
    c = lax.axis_index("c")
    s = lax.axis_index("s")
    w = c * NS + s

    def _zero_row(i, _):
        for k in range(D // 16):
            xbuf[0, i, pl.ds(k * 16, 16)] = jnp.zeros((16,), jnp.float32)
        return 0
    lax.fori_loop(0, CHUNK, _zero_row, 0)
    zbuf = xbuf.at[0]

    @pl.when(s < NS - 1)
    def _():
        for k in range(ROWS_BIG // CHUNK):
            off = pl.multiple_of(s * ROWS_BIG + k * CHUNK, 8)
            pltpu.sync_copy(zbuf, acc.at[pl.ds(off, CHUNK)])

    @pl.when(s == NS - 1)
    def _():
        for k in range(ROWS_LAST // CHUNK):
            pltpu.sync_copy(
                zbuf, acc.at[pl.ds((NS - 1) * ROWS_BIG + k * CHUNK, CHUNK)])

    roff = pl.multiple_of(s * RELPT, 8)
    pltpu.sync_copy(rel_hbm.at[pl.ds(roff, RELPT)], zbuf.at[pl.ds(0, RELPT)])
    for i in range(RELPT):
        for k in range(D // 16):
            xbuf[0, i, pl.ds(k * 16, 16)] = -xbuf[0, i, pl.ds(k * 16, 16)]
    pltpu.sync_copy(zbuf.at[pl.ds(0, RELPT)], negrel.at[pl.ds(roff, RELPT)])

    plsc.subcore_barrier()

    def _batch(b, _):
        pltpu.sync_copy(src_hbm.at[w, b], srcb)
        pltpu.sync_copy(dst_hbm.at[w, b], dstb)
        pltpu.sync_copy(et_hbm.at[w, b], etb)
        pltpu.async_copy(x_hbm.at[srcb.at[0]], xbuf.at[0], semx.at[0])

        def _chunk(jj, _):
            p = lax.rem(jj, 2)
            q = 1 - p

            for h in range(2):
                @pl.when(jj > 0)
                def _():
                    pltpu.make_async_copy(
                        rbuf.at[h], acc.at[dstb.at[jj - 1, pl.ds(h * HC, HC)]],
                        semrs.at[h]).wait()
                pltpu.async_copy(negrel.at[etb.at[jj, pl.ds(h * HC, HC)]],
                                 rbuf.at[h], semr.at[h])

            @pl.when(jj > 0)
            def _():
                pltpu.make_async_copy(xbuf.at[q], acc.at[dstb.at[jj - 1]],
                                      semsx.at[q]).wait()

            @pl.when(jj < BCH - 1)
            def _():
                pltpu.async_copy(x_hbm.at[srcb.at[jj + 1]], xbuf.at[q],
                                 semx.at[q])

            pltpu.make_async_copy(x_hbm.at[srcb.at[jj]], xbuf.at[p],
                                  semx.at[p]).wait()
            pltpu.async_copy(xbuf.at[p], acc.at[dstb.at[jj]], semsx.at[p],
                             add=True)

            for h in range(2):
                pltpu.make_async_copy(negrel.at[etb.at[jj, pl.ds(h * HC, HC)]],
                                      rbuf.at[h], semr.at[h]).wait()
                pltpu.async_copy(rbuf.at[h],
                                 acc.at[dstb.at[jj, pl.ds(h * HC, HC)]],
                                 semrs.at[h], add=True)
            return 0
        lax.fori_loop(0, BCH, _chunk, 0)
        pltpu.make_async_copy(xbuf.at[(BCH - 1) % 2],
                              acc.at[dstb.at[BCH - 1]],
                              semsx.at[(BCH - 1) % 2]).wait()
        for h in range(2):
            pltpu.make_async_copy(
                rbuf.at[h], acc.at[dstb.at[BCH - 1, pl.ds(h * HC, HC)]],
                semrs.at[h]).wait()
        return 0
    lax.fori_loop(0, NBATCH, _batch, 0)

    plsc.subcore_barrier()

    @pl.when(s < NS - 1)
    def _():
        off = pl.multiple_of(s * ROWS_BIG, 8)
        pltpu.sync_copy(acc.at[pl.ds(off, ROWS_BIG)],
                        out_hbm.at[c, pl.ds(off, ROWS_BIG)])

    @pl.when(s == NS - 1)
    def _():
        off = (NS - 1) * ROWS_BIG
        pltpu.sync_copy(acc.at[pl.ds(off, ROWS_LAST)],
                        out_hbm.at[c, pl.ds(off, ROWS_LAST)])


_sc_agg = pl.kernel(
    _sc_body,
    mesh=plsc.VectorSubcoreMesh(core_axis_name="c", subcore_axis_name="s"),
    out_type=jax.ShapeDtypeStruct((NC, N_NODES, D), jnp.float32),
    scratch_types=[
        pltpu.VMEM((BCH, CHUNK), jnp.int32),
        pltpu.VMEM((BCH, CHUNK), jnp.int32),
        pltpu.VMEM((BCH, CHUNK), jnp.int32),
        pltpu.VMEM((2, CHUNK, D), jnp.float32),
        pltpu.VMEM((2, HC, D), jnp.float32),
        pltpu.VMEM_SHARED((N_NODES, D), jnp.float32),
        pltpu.VMEM_SHARED((N_REL, D), jnp.float32),
        pltpu.SemaphoreType.DMA((2,)),
        pltpu.SemaphoreType.DMA((2,)),
        pltpu.SemaphoreType.DMA((2,)),
        pltpu.SemaphoreType.DMA((2,)),
    ],
)


BN = 2000


def _tc_body(x_ref, part_ref, rel_ref, ws_ref, wf_ref, wr_ref, b_ref,
             out_ref, relout_ref):
    i = pl.program_id(0)
    dn = (((1,), (1,)), ((), ()))
    agg = part_ref[0] + part_ref[1]
    out_ref[...] = (
        lax.dot_general(x_ref[...], ws_ref[...], dn,
                        preferred_element_type=jnp.float32)
        + lax.dot_general(agg, wf_ref[...], dn,
                          preferred_element_type=jnp.float32)
        + b_ref[...]
    )

    @pl.when(i == 0)
    def _():
        relout_ref[...] = lax.dot_general(rel_ref[...], wr_ref[...], dn,
                                          preferred_element_type=jnp.float32)


def _tc_finish(x, part, rel_emb, W_self, W_forward, W_rel, bias2d):
    return pl.pallas_call(
        _tc_body,
        grid=(N_NODES // BN,),
        in_specs=[
            pl.BlockSpec((BN, D), lambda i: (i, 0)),
            pl.BlockSpec((NC, BN, D), lambda i: (0, i, 0)),
            pl.BlockSpec((N_REL, D), lambda i: (0, 0)),
            pl.BlockSpec((D, D), lambda i: (0, 0)),
            pl.BlockSpec((D, D), lambda i: (0, 0)),
            pl.BlockSpec((D, D), lambda i: (0, 0)),
            pl.BlockSpec((1, D), lambda i: (0, 0)),
        ],
        out_specs=[
            pl.BlockSpec((BN, D), lambda i: (i, 0)),
            pl.BlockSpec((N_REL, D), lambda i: (0, 0)),
        ],
        out_shape=[
            jax.ShapeDtypeStruct((N_NODES, D), jnp.float32),
            jax.ShapeDtypeStruct((N_REL, D), jnp.float32),
        ],
    )(x, part, rel_emb, W_self, W_forward, W_rel, bias2d)


def kernel(x, edge_index, edge_type, rel_emb, W_self, W_forward, W_rel, bias):
    e4 = (NW, NBATCH, BCH, CHUNK)
    part = _sc_agg(x, edge_index[0].reshape(e4), edge_index[1].reshape(e4),
                   edge_type.reshape(e4), rel_emb)
    out, rel_out = _tc_finish(x, part, rel_emb, W_self, W_forward, W_rel,
                              bias.reshape(1, D))
    return out, rel_out


# speedup vs baseline: 9.7803x; 1.2398x over previous
import jax
import jax.numpy as jnp
from jax import lax
from jax.experimental import pallas as pl
from jax.experimental.pallas import tpu as pltpu
from jax.experimental.pallas import tpu_sc as plsc

N_NODES = 10000
N_EDGES = 320000
D = 128
N_REL = 256

NC = 2
NS = 16
NW = NC * NS
EPW = N_EDGES // NW
CHUNK = 80
NCHUNK = EPW // CHUNK
NBATCH = 5
BCH = NCHUNK // NBATCH
HC = CHUNK // 2
RELPT = N_REL // NS
ROWS_BIG = 640
ROWS_LAST = N_NODES - (NS - 1) * ROWS_BIG


def _sc_body(x_hbm, src_hbm, dst_hbm, et_hbm, rel_hbm, out_hbm,
             srcb, etb, dstb, xbuf, rbuf, acc, negrel, semx, semsx, semr):
    c = lax.axis_index("c")
    s = lax.axis_index("s")
    w = c * NS + s

    def _zero_row(i, _):
        for k in range(D // 16):
            xbuf[0, i, pl.ds(k * 16, 16)] = jnp.zeros((16,), jnp.float32)
        return 0
    lax.fori_loop(0, CHUNK, _zero_row, 0)
    zbuf = xbuf.at[0]

    @pl.when(s < NS - 1)
    def _():
        for k in range(ROWS_BIG // CHUNK):
            off = pl.multiple_of(s * ROWS_BIG + k * CHUNK, 8)
            pltpu.sync_copy(zbuf, acc.at[pl.ds(off, CHUNK)])

    @pl.when(s == NS - 1)
    def _():
        for k in range(ROWS_LAST // CHUNK):
            pltpu.sync_copy(
                zbuf, acc.at[pl.ds((NS - 1) * ROWS_BIG + k * CHUNK, CHUNK)])

    roff = pl.multiple_of(s * RELPT, 8)
    pltpu.sync_copy(rel_hbm.at[pl.ds(roff, RELPT)], negrel.at[pl.ds(roff, RELPT)])

    plsc.subcore_barrier()

    def _batch(b, _):
        pltpu.sync_copy(src_hbm.at[w, b], srcb)
        pltpu.sync_copy(dst_hbm.at[w, b], dstb)
        pltpu.sync_copy(et_hbm.at[w, b], etb)
        pltpu.async_copy(x_hbm.at[srcb.at[0]], xbuf.at[0], semx.at[0])

        def _chunk(jj, _):
            p = lax.rem(jj, 2)
            q = 1 - p

            for h in range(2):
                pltpu.async_copy(negrel.at[etb.at[jj, pl.ds(h * HC, HC)]],
                                 rbuf.at[h], semr.at[h])

            @pl.when(jj > 0)
            def _():
                pltpu.make_async_copy(xbuf.at[q], acc.at[dstb.at[jj - 1]],
                                      semsx.at[q]).wait()

            @pl.when(jj < BCH - 1)
            def _():
                pltpu.async_copy(x_hbm.at[srcb.at[jj + 1]], xbuf.at[q],
                                 semx.at[q])

            pltpu.make_async_copy(x_hbm.at[srcb.at[jj]], xbuf.at[p],
                                  semx.at[p]).wait()

            for h in range(2):
                pltpu.make_async_copy(negrel.at[etb.at[jj, pl.ds(h * HC, HC)]],
                                      rbuf.at[h], semr.at[h]).wait()

                for pp in range(2):
                    @pl.when(p == pp)
                    def _():
                        def _vrows(r, _):
                            for u in range(4):
                                rr = h * HC + r * 4 + u
                                for k in range(D // 16):
                                    sl = pl.ds(k * 16, 16)
                                    xbuf[pp, rr, sl] = (
                                        xbuf[pp, rr, sl]
                                        - rbuf[h, r * 4 + u, sl])
                            return 0
                        lax.fori_loop(0, HC // 4, _vrows, 0)

            pltpu.async_copy(xbuf.at[p], acc.at[dstb.at[jj]], semsx.at[p],
                             add=True)
            return 0
        lax.fori_loop(0, BCH, _chunk, 0)
        pltpu.make_async_copy(xbuf.at[(BCH - 1) % 2],
                              acc.at[dstb.at[BCH - 1]],
                              semsx.at[(BCH - 1) % 2]).wait()
        return 0
    lax.fori_loop(0, NBATCH, _batch, 0)

    plsc.subcore_barrier()

    @pl.when(s < NS - 1)
    def _():
        off = pl.multiple_of(s * ROWS_BIG, 8)
        pltpu.sync_copy(acc.at[pl.ds(off, ROWS_BIG)],
                        out_hbm.at[c, pl.ds(off, ROWS_BIG)])

    @pl.when(s == NS - 1)
    def _():
        off = (NS - 1) * ROWS_BIG
        pltpu.sync_copy(acc.at[pl.ds(off, ROWS_LAST)],
                        out_hbm.at[c, pl.ds(off, ROWS_LAST)])


_sc_agg = pl.kernel(
    _sc_body,
    mesh=plsc.VectorSubcoreMesh(core_axis_name="c", subcore_axis_name="s"),
    out_type=jax.ShapeDtypeStruct((NC, N_NODES, D), jnp.float32),
    scratch_types=[
        pltpu.VMEM((BCH, CHUNK), jnp.int32),
        pltpu.VMEM((BCH, CHUNK), jnp.int32),
        pltpu.VMEM((BCH, CHUNK), jnp.int32),
        pltpu.VMEM((2, CHUNK, D), jnp.float32),
        pltpu.VMEM((2, HC, D), jnp.float32),
        pltpu.VMEM_SHARED((N_NODES, D), jnp.float32),
        pltpu.VMEM_SHARED((N_REL, D), jnp.float32),
        pltpu.SemaphoreType.DMA((2,)),
        pltpu.SemaphoreType.DMA((2,)),
        pltpu.SemaphoreType.DMA((2,)),
    ],
)


BN = 2000


def _tc_body(x_ref, part_ref, rel_ref, ws_ref, wf_ref, wr_ref, b_ref,
             out_ref, relout_ref):
    i = pl.program_id(0)
    dn = (((1,), (1,)), ((), ()))
    agg = part_ref[0] + part_ref[1]
    out_ref[...] = (
        lax.dot_general(x_ref[...], ws_ref[...], dn,
                        preferred_element_type=jnp.float32)
        + lax.dot_general(agg, wf_ref[...], dn,
                          preferred_element_type=jnp.float32)
        + b_ref[...]
    )

    @pl.when(i == 0)
    def _():
        relout_ref[...] = lax.dot_general(rel_ref[...], wr_ref[...], dn,
                                          preferred_element_type=jnp.float32)


def _tc_finish(x, part, rel_emb, W_self, W_forward, W_rel, bias2d):
    return pl.pallas_call(
        _tc_body,
        grid=(N_NODES // BN,),
        in_specs=[
            pl.BlockSpec((BN, D), lambda i: (i, 0)),
            pl.BlockSpec((NC, BN, D), lambda i: (0, i, 0)),
            pl.BlockSpec((N_REL, D), lambda i: (0, 0)),
            pl.BlockSpec((D, D), lambda i: (0, 0)),
            pl.BlockSpec((D, D), lambda i: (0, 0)),
            pl.BlockSpec((D, D), lambda i: (0, 0)),
            pl.BlockSpec((1, D), lambda i: (0, 0)),
        ],
        out_specs=[
            pl.BlockSpec((BN, D), lambda i: (i, 0)),
            pl.BlockSpec((N_REL, D), lambda i: (0, 0)),
        ],
        out_shape=[
            jax.ShapeDtypeStruct((N_NODES, D), jnp.float32),
            jax.ShapeDtypeStruct((N_REL, D), jnp.float32),
        ],
    )(x, part, rel_emb, W_self, W_forward, W_rel, bias2d)


def kernel(x, edge_index, edge_type, rel_emb, W_self, W_forward, W_rel, bias):
    e4 = (NW, NBATCH, BCH, CHUNK)
    part = _sc_agg(x, edge_index[0].reshape(e4), edge_index[1].reshape(e4),
                   edge_type.reshape(e4), rel_emb)
    out, rel_out = _tc_finish(x, part, rel_emb, W_self, W_forward, W_rel,
                              bias.reshape(1, D))
    return out, rel_out


# speedup vs baseline: 9.8074x; 1.0028x over previous
import jax
import jax.numpy as jnp
from jax import lax
from jax.experimental import pallas as pl
from jax.experimental.pallas import tpu as pltpu
from jax.experimental.pallas import tpu_sc as plsc

N_NODES = 10000
N_EDGES = 320000
D = 128
N_REL = 256

NC = 2
NS = 16
NW = NC * NS
EPW = N_EDGES // NW
CHUNK = 80
NCHUNK = EPW // CHUNK
NBATCH = 5
BCH = NCHUNK // NBATCH
HC = CHUNK // 2
RELPT = N_REL // NS
ROWS_BIG = 640
ROWS_LAST = N_NODES - (NS - 1) * ROWS_BIG


def _sc_body(x_hbm, src_hbm, dst_hbm, et_hbm, rel_hbm, out_hbm,
             srcb, etb, dstb, xbuf, rbuf, acc, negrel, semx, semsx, semr):
    c = lax.axis_index("c")
    s = lax.axis_index("s")
    w = c * NS + s

    def _zero_row(i, _):
        for k in range(D // 16):
            xbuf[0, i, pl.ds(k * 16, 16)] = jnp.zeros((16,), jnp.float32)
        return 0
    lax.fori_loop(0, CHUNK, _zero_row, 0)
    zbuf = xbuf.at[0]

    @pl.when(s < NS - 1)
    def _():
        for k in range(ROWS_BIG // CHUNK):
            off = pl.multiple_of(s * ROWS_BIG + k * CHUNK, 8)
            pltpu.sync_copy(zbuf, acc.at[pl.ds(off, CHUNK)])

    @pl.when(s == NS - 1)
    def _():
        for k in range(ROWS_LAST // CHUNK):
            pltpu.sync_copy(
                zbuf, acc.at[pl.ds((NS - 1) * ROWS_BIG + k * CHUNK, CHUNK)])

    roff = pl.multiple_of(s * RELPT, 8)
    pltpu.sync_copy(rel_hbm.at[pl.ds(roff, RELPT)], negrel.at[pl.ds(roff, RELPT)])

    plsc.subcore_barrier()

    def _batch(b, _):
        pltpu.sync_copy(src_hbm.at[w, b], srcb)
        pltpu.sync_copy(dst_hbm.at[w, b], dstb)
        pltpu.sync_copy(et_hbm.at[w, b], etb)
        pltpu.async_copy(x_hbm.at[srcb.at[0]], xbuf.at[0], semx.at[0])

        def _chunk(jj, _):
            p = lax.rem(jj, 2)
            q = 1 - p

            for h in range(2):
                pltpu.async_copy(negrel.at[etb.at[jj, pl.ds(h * HC, HC)]],
                                 rbuf.at[h], semr.at[h])

            @pl.when(jj > 0)
            def _():
                pltpu.make_async_copy(xbuf.at[q], acc.at[dstb.at[jj - 1]],
                                      semsx.at[q]).wait()

            @pl.when(jj < BCH - 1)
            def _():
                pltpu.async_copy(x_hbm.at[srcb.at[jj + 1]], xbuf.at[q],
                                 semx.at[q])

            pltpu.make_async_copy(x_hbm.at[srcb.at[jj]], xbuf.at[p],
                                  semx.at[p]).wait()

            for h in range(2):
                pltpu.make_async_copy(negrel.at[etb.at[jj, pl.ds(h * HC, HC)]],
                                      rbuf.at[h], semr.at[h]).wait()

                for pp in range(2):
                    @pl.when(p == pp)
                    def _():
                        def _vrows(r, _):
                            for u in range(4):
                                rr = h * HC + r * 4 + u
                                for k in range(D // 16):
                                    sl = pl.ds(k * 16, 16)
                                    xbuf[pp, rr, sl] = (
                                        xbuf[pp, rr, sl]
                                        - rbuf[h, r * 4 + u, sl])
                            return 0
                        lax.fori_loop(0, HC // 4, _vrows, 0)

            pltpu.async_copy(xbuf.at[p], acc.at[dstb.at[jj]], semsx.at[p],
                             add=True)
            return 0
        lax.fori_loop(0, BCH, _chunk, 0)
        pltpu.make_async_copy(xbuf.at[(BCH - 1) % 2],
                              acc.at[dstb.at[BCH - 1]],
                              semsx.at[(BCH - 1) % 2]).wait()
        return 0
    lax.fori_loop(0, NBATCH, _batch, 0)

    plsc.subcore_barrier()

    @pl.when(s < NS - 1)
    def _():
        off = pl.multiple_of(s * ROWS_BIG, 8)
        pltpu.sync_copy(acc.at[pl.ds(off, ROWS_BIG)],
                        out_hbm.at[c, pl.ds(off, ROWS_BIG)])

    @pl.when(s == NS - 1)
    def _():
        off = (NS - 1) * ROWS_BIG
        pltpu.sync_copy(acc.at[pl.ds(off, ROWS_LAST)],
                        out_hbm.at[c, pl.ds(off, ROWS_LAST)])


_sc_agg = pl.kernel(
    _sc_body,
    mesh=plsc.VectorSubcoreMesh(core_axis_name="c", subcore_axis_name="s"),
    out_type=jax.ShapeDtypeStruct((NC, N_NODES, D), jnp.float32),
    scratch_types=[
        pltpu.VMEM((BCH, CHUNK), jnp.int32),
        pltpu.VMEM((BCH, CHUNK), jnp.int32),
        pltpu.VMEM((BCH, CHUNK), jnp.int32),
        pltpu.VMEM((2, CHUNK, D), jnp.float32),
        pltpu.VMEM((2, HC, D), jnp.float32),
        pltpu.VMEM_SHARED((N_NODES, D), jnp.float32),
        pltpu.VMEM_SHARED((N_REL, D), jnp.float32),
        pltpu.SemaphoreType.DMA((2,)),
        pltpu.SemaphoreType.DMA((2,)),
        pltpu.SemaphoreType.DMA((2,)),
    ],
)


BN = 2000


def _tc1_body(x_ref, rel_ref, ws_ref, wr_ref, b_ref, y0_ref, relout_ref):
    i = pl.program_id(0)
    dn = (((1,), (1,)), ((), ()))
    y0_ref[...] = lax.dot_general(x_ref[...], ws_ref[...], dn,
                                  preferred_element_type=jnp.float32) + b_ref[...]

    @pl.when(i == 0)
    def _():
        relout_ref[...] = lax.dot_general(rel_ref[...], wr_ref[...], dn,
                                          preferred_element_type=jnp.float32)


def _tc1(x, rel_emb, W_self, W_rel, bias2d):
    return pl.pallas_call(
        _tc1_body,
        grid=(N_NODES // BN,),
        in_specs=[
            pl.BlockSpec((BN, D), lambda i: (i, 0)),
            pl.BlockSpec((N_REL, D), lambda i: (0, 0)),
            pl.BlockSpec((D, D), lambda i: (0, 0)),
            pl.BlockSpec((D, D), lambda i: (0, 0)),
            pl.BlockSpec((1, D), lambda i: (0, 0)),
        ],
        out_specs=[
            pl.BlockSpec((BN, D), lambda i: (i, 0)),
            pl.BlockSpec((N_REL, D), lambda i: (0, 0)),
        ],
        out_shape=[
            jax.ShapeDtypeStruct((N_NODES, D), jnp.float32),
            jax.ShapeDtypeStruct((N_REL, D), jnp.float32),
        ],
    )(x, rel_emb, W_self, W_rel, bias2d)


def _tc2_body(y0_ref, part_ref, wf_ref, out_ref):
    dn = (((1,), (1,)), ((), ()))
    agg = part_ref[0] + part_ref[1]
    out_ref[...] = y0_ref[...] + lax.dot_general(
        agg, wf_ref[...], dn, preferred_element_type=jnp.float32)


def _tc2(y0, part, W_forward):
    return pl.pallas_call(
        _tc2_body,
        grid=(N_NODES // BN,),
        in_specs=[
            pl.BlockSpec((BN, D), lambda i: (i, 0)),
            pl.BlockSpec((NC, BN, D), lambda i: (0, i, 0)),
            pl.BlockSpec((D, D), lambda i: (0, 0)),
        ],
        out_specs=pl.BlockSpec((BN, D), lambda i: (i, 0)),
        out_shape=jax.ShapeDtypeStruct((N_NODES, D), jnp.float32),
    )(y0, part, W_forward)


def kernel(x, edge_index, edge_type, rel_emb, W_self, W_forward, W_rel, bias):
    e4 = (NW, NBATCH, BCH, CHUNK)
    y0, rel_out = _tc1(x, rel_emb, W_self, W_rel, bias.reshape(1, D))
    part = _sc_agg(x, edge_index[0].reshape(e4), edge_index[1].reshape(e4),
                   edge_type.reshape(e4), rel_emb)
    out = _tc2(y0, part, W_forward)
    return out, rel_out


# speedup vs baseline: 11.5988x; 1.1827x over previous
import jax
import jax.numpy as jnp
from jax import lax
from jax.experimental import pallas as pl
from jax.experimental.pallas import tpu as pltpu
from jax.experimental.pallas import tpu_sc as plsc

N_NODES = 10000
N_EDGES = 320000
D = 128
N_REL = 256

NC = 2
NS = 16
NW = NC * NS
EPW = N_EDGES // NW
CHUNK = 80
NCHUNK = EPW // CHUNK
NBATCH = 5
BCH = NCHUNK // NBATCH
HC = CHUNK // 2
RELPT = N_REL // NS
ROWS_BIG = 640
ROWS_LAST = N_NODES - (NS - 1) * ROWS_BIG


def _sc_body(x_hbm, src_hbm, dst_hbm, et_hbm, rel_hbm, out_hbm,
             srcb, etb, dstb, xbuf, rbuf, relb16, acc, negrel, semx, semsx,
             semr):
    c = lax.axis_index("c")
    s = lax.axis_index("s")
    w = c * NS + s

    def _zero_row(i, _):
        for k in range(D // 16):
            xbuf[0, i, pl.ds(k * 16, 16)] = jnp.zeros((16,), jnp.float32)
        return 0
    lax.fori_loop(0, CHUNK, _zero_row, 0)
    zbuf = xbuf.at[0]

    @pl.when(s < NS - 1)
    def _():
        for k in range(ROWS_BIG // CHUNK):
            off = pl.multiple_of(s * ROWS_BIG + k * CHUNK, 8)
            pltpu.sync_copy(zbuf, acc.at[pl.ds(off, CHUNK)])

    @pl.when(s == NS - 1)
    def _():
        for k in range(ROWS_LAST // CHUNK):
            pltpu.sync_copy(
                zbuf, acc.at[pl.ds((NS - 1) * ROWS_BIG + k * CHUNK, CHUNK)])

    roff = pl.multiple_of(s * RELPT, 8)
    pltpu.sync_copy(rel_hbm.at[pl.ds(roff, RELPT)], zbuf.at[pl.ds(0, RELPT)])
    rnd = jnp.full((16,), 0x8000, jnp.int32)
    himask = jnp.full((16,), -65536, jnp.int32)
    for i in range(RELPT):
        for k in range(D // 32):
            a = xbuf[0, i, pl.ds(k * 32, 16)]
            b = xbuf[0, i, pl.ds(k * 32 + 16, 16)]
            pa = lax.shift_right_logical(
                lax.bitcast_convert_type(a, jnp.int32) + rnd,
                jnp.full((16,), 16, jnp.int32))
            pb = lax.bitwise_and(
                lax.bitcast_convert_type(b, jnp.int32) + rnd, himask)
            relb16[i, pl.ds(k * 16, 16)] = lax.bitwise_or(pa, pb)
    pltpu.sync_copy(relb16, negrel.at[pl.ds(roff, RELPT)])

    plsc.subcore_barrier()

    def _batch(b, _):
        pltpu.sync_copy(src_hbm.at[w, b], srcb)
        pltpu.sync_copy(dst_hbm.at[w, b], dstb)
        pltpu.sync_copy(et_hbm.at[w, b], etb)
        pltpu.async_copy(x_hbm.at[srcb.at[0]], xbuf.at[0], semx.at[0])

        def _chunk(jj, _):
            p = lax.rem(jj, 2)
            q = 1 - p

            for h in range(2):
                pltpu.async_copy(negrel.at[etb.at[jj, pl.ds(h * HC, HC)]],
                                 rbuf.at[h], semr.at[h])

            @pl.when(jj > 0)
            def _():
                pltpu.make_async_copy(xbuf.at[q], acc.at[dstb.at[jj - 1]],
                                      semsx.at[q]).wait()

            @pl.when(jj < BCH - 1)
            def _():
                pltpu.async_copy(x_hbm.at[srcb.at[jj + 1]], xbuf.at[q],
                                 semx.at[q])

            pltpu.make_async_copy(x_hbm.at[srcb.at[jj]], xbuf.at[p],
                                  semx.at[p]).wait()

            for h in range(2):
                pltpu.make_async_copy(negrel.at[etb.at[jj, pl.ds(h * HC, HC)]],
                                      rbuf.at[h], semr.at[h]).wait()

                for pp in range(2):
                    @pl.when(p == pp)
                    def _():
                        sh16 = jnp.full((16,), 16, jnp.int32)
                        himask = jnp.full((16,), -65536, jnp.int32)

                        def _vrows(r, _):
                            for u in range(4):
                                rr = h * HC + r * 4 + u
                                for k in range(D // 32):
                                    w = rbuf[h, r * 4 + u, pl.ds(k * 16, 16)]
                                    ra = lax.bitcast_convert_type(
                                        lax.shift_left(w, sh16), jnp.float32)
                                    rb = lax.bitcast_convert_type(
                                        lax.bitwise_and(w, himask),
                                        jnp.float32)
                                    sa = pl.ds(k * 32, 16)
                                    sb = pl.ds(k * 32 + 16, 16)
                                    xbuf[pp, rr, sa] = xbuf[pp, rr, sa] - ra
                                    xbuf[pp, rr, sb] = xbuf[pp, rr, sb] - rb
                            return 0
                        lax.fori_loop(0, HC // 4, _vrows, 0)

            pltpu.async_copy(xbuf.at[p], acc.at[dstb.at[jj]], semsx.at[p],
                             add=True)
            return 0
        lax.fori_loop(0, BCH, _chunk, 0)
        pltpu.make_async_copy(xbuf.at[(BCH - 1) % 2],
                              acc.at[dstb.at[BCH - 1]],
                              semsx.at[(BCH - 1) % 2]).wait()
        return 0
    lax.fori_loop(0, NBATCH, _batch, 0)

    plsc.subcore_barrier()

    @pl.when(s < NS - 1)
    def _():
        off = pl.multiple_of(s * ROWS_BIG, 8)
        pltpu.sync_copy(acc.at[pl.ds(off, ROWS_BIG)],
                        out_hbm.at[c, pl.ds(off, ROWS_BIG)])

    @pl.when(s == NS - 1)
    def _():
        off = (NS - 1) * ROWS_BIG
        pltpu.sync_copy(acc.at[pl.ds(off, ROWS_LAST)],
                        out_hbm.at[c, pl.ds(off, ROWS_LAST)])


_sc_agg = pl.kernel(
    _sc_body,
    mesh=plsc.VectorSubcoreMesh(core_axis_name="c", subcore_axis_name="s"),
    out_type=jax.ShapeDtypeStruct((NC, N_NODES, D), jnp.float32),
    scratch_types=[
        pltpu.VMEM((BCH, CHUNK), jnp.int32),
        pltpu.VMEM((BCH, CHUNK), jnp.int32),
        pltpu.VMEM((BCH, CHUNK), jnp.int32),
        pltpu.VMEM((2, CHUNK, D), jnp.float32),
        pltpu.VMEM((2, HC, D // 2), jnp.int32),
        pltpu.VMEM((RELPT, D // 2), jnp.int32),
        pltpu.VMEM_SHARED((N_NODES, D), jnp.float32),
        pltpu.VMEM_SHARED((N_REL, D // 2), jnp.int32),
        pltpu.SemaphoreType.DMA((2,)),
        pltpu.SemaphoreType.DMA((2,)),
        pltpu.SemaphoreType.DMA((2,)),
    ],
)


BN = 2000


def _tc1_body(x_ref, rel_ref, ws_ref, wr_ref, b_ref, y0_ref, relout_ref):
    i = pl.program_id(0)
    dn = (((1,), (1,)), ((), ()))
    y0_ref[...] = lax.dot_general(x_ref[...], ws_ref[...], dn,
                                  preferred_element_type=jnp.float32) + b_ref[...]

    @pl.when(i == 0)
    def _():
        relout_ref[...] = lax.dot_general(rel_ref[...], wr_ref[...], dn,
                                          preferred_element_type=jnp.float32)


def _tc1(x, rel_emb, W_self, W_rel, bias2d):
    return pl.pallas_call(
        _tc1_body,
        grid=(N_NODES // BN,),
        in_specs=[
            pl.BlockSpec((BN, D), lambda i: (i, 0)),
            pl.BlockSpec((N_REL, D), lambda i: (0, 0)),
            pl.BlockSpec((D, D), lambda i: (0, 0)),
            pl.BlockSpec((D, D), lambda i: (0, 0)),
            pl.BlockSpec((1, D), lambda i: (0, 0)),
        ],
        out_specs=[
            pl.BlockSpec((BN, D), lambda i: (i, 0)),
            pl.BlockSpec((N_REL, D), lambda i: (0, 0)),
        ],
        out_shape=[
            jax.ShapeDtypeStruct((N_NODES, D), jnp.float32),
            jax.ShapeDtypeStruct((N_REL, D), jnp.float32),
        ],
    )(x, rel_emb, W_self, W_rel, bias2d)


def _tc2_body(y0_ref, part_ref, wf_ref, out_ref):
    dn = (((1,), (1,)), ((), ()))
    agg = part_ref[0] + part_ref[1]
    out_ref[...] = y0_ref[...] + lax.dot_general(
        agg, wf_ref[...], dn, preferred_element_type=jnp.float32)


def _tc2(y0, part, W_forward):
    return pl.pallas_call(
        _tc2_body,
        grid=(N_NODES // BN,),
        in_specs=[
            pl.BlockSpec((BN, D), lambda i: (i, 0)),
            pl.BlockSpec((NC, BN, D), lambda i: (0, i, 0)),
            pl.BlockSpec((D, D), lambda i: (0, 0)),
        ],
        out_specs=pl.BlockSpec((BN, D), lambda i: (i, 0)),
        out_shape=jax.ShapeDtypeStruct((N_NODES, D), jnp.float32),
    )(y0, part, W_forward)


def kernel(x, edge_index, edge_type, rel_emb, W_self, W_forward, W_rel, bias):
    e4 = (NW, NBATCH, BCH, CHUNK)
    y0, rel_out = _tc1(x, rel_emb, W_self, W_rel, bias.reshape(1, D))
    part = _sc_agg(x, edge_index[0].reshape(e4), edge_index[1].reshape(e4),
                   edge_type.reshape(e4), rel_emb)
    out = _tc2(y0, part, W_forward)
    return out, rel_out
